# Initial kernel scaffold; baseline (speedup 1.0000x reference)
import jax
import jax.numpy as jnp
from jax.experimental import pallas as pl


def kernel(P_Q_inj, senders, receivers, edge_features, W_in, b_in, We0, be0, Wn0, bn0, Wd0, bd0, We1, be1, Wn1, bn1, Wd1, bd1, We2, be2, Wn2, bn2, Wd2, bd2):
    raise NotImplementedError("write your pallas kernel here")


# speedup vs baseline: 4.1768x; 4.1768x over previous
import functools

import jax
import jax.numpy as jnp
from jax import lax
from jax.experimental import pallas as pl
from jax.experimental.pallas import tpu as pltpu
from jax.experimental.pallas import tpu_sc as plsc

H = 32
D_EDGE = 4
NODE_BLK = 512
EDGE_BLK = 4096
IDX_W = 128
CHUNK_ROWS = 4
NUM_SC = 2
NUM_SUBCORES = 16


def _ceil_to(x, m):
    return (x + m - 1) // m * m


def _dot(x, w):
    return jnp.dot(x, w, preferred_element_type=jnp.float32)


def _prologue_body(pq, win, bin_, wsv, wsh, wrv, wrh, be, h_ref, a_ref, b_ref):
    h = _dot(pq[...], win[...]) + bin_[...]
    h_ref[...] = h
    a_ref[...] = _dot(h, wsh[...]) + wsv[...][0:1, :]
    b_ref[...] = _dot(h, wrh[...]) + wrv[...][0:1, :] + be[...]


def _prologue(npad, pq, win, bin_, wsv, wsh, wrv, wrh, be):
    grid = npad // NODE_BLK
    full = lambda shape: pl.BlockSpec(shape, lambda i: (0, 0))
    row = lambda w: pl.BlockSpec((NODE_BLK, w), lambda i: (i, 0))
    out = jax.ShapeDtypeStruct((npad, H), jnp.float32)
    return pl.pallas_call(
        _prologue_body,
        grid=(grid,),
        in_specs=[row(2), full((2, H)), full((1, H)), full((2, H)),
                  full((H, H)), full((2, H)), full((H, H)), full((1, H))],
        out_specs=[row(H), row(H), row(H)],
        out_shape=[out, out, out],
    )(pq, win, bin_, wsv, wsh, wrv, wrh, be)


def _edge_c_body(ef, we, c_ref):
    c_ref[...] = _dot(ef[...], we[...])


def _edge_c(epad, ef, we):
    grid = epad // EDGE_BLK
    return pl.pallas_call(
        _edge_c_body,
        grid=(grid,),
        in_specs=[pl.BlockSpec((EDGE_BLK, D_EDGE), lambda i: (i, 0)),
                  pl.BlockSpec((D_EDGE, H), lambda i: (0, 0))],
        out_specs=pl.BlockSpec((EDGE_BLK, H), lambda i: (i, 0)),
        out_shape=jax.ShapeDtypeStruct((epad, H), jnp.float32),
    )(ef, we)


def _node_body(vp, h, p0, p1, wnv, wnh, wna, bn, wd, bd,
               wsv, wsh, wrv, wrh, be,
               vn_ref, hn_ref, an_ref, bn_ref):
    agg = p0[...] + p1[...]
    vo = (_dot(vp[...], wnv[...]) + _dot(h[...], wnh[...])
          + _dot(agg, wna[...]) + bn[...])
    hn = jnp.maximum(vo, 0.0)
    vn = vp[...] + _dot(hn, wd[...]) + bd[...]
    vn_ref[...] = vn
    hn_ref[...] = hn
    an_ref[...] = _dot(vn, wsv[...]) + _dot(hn, wsh[...])
    bn_ref[...] = _dot(vn, wrv[...]) + _dot(hn, wrh[...]) + be[...]


def _node_update(npad, vp, h, p0, p1, wnv, wnh, wna, bn, wd, bd,
                 wsv, wsh, wrv, wrh, be):
    grid = npad // NODE_BLK
    full = lambda shape: pl.BlockSpec(shape, lambda i: (0, 0))
    row = lambda w: pl.BlockSpec((NODE_BLK, w), lambda i: (i, 0))
    outh = jax.ShapeDtypeStruct((npad, H), jnp.float32)
    outv = jax.ShapeDtypeStruct((npad, 2), jnp.float32)
    return pl.pallas_call(
        _node_body,
        grid=(grid,),
        in_specs=[row(2), row(H), row(H), row(H),
                  full((2, H)), full((H, H)), full((H, H)), full((1, H)),
                  full((H, 2)), full((1, 2)),
                  full((2, H)), full((H, H)), full((2, H)), full((H, H)),
                  full((1, H))],
        out_specs=[row(2), row(H), row(H), row(H)],
        out_shape=[outv, outh, outh, outh],
    )(vp, h, p0, p1, wnv, wnh, wna, bn, wd, bd, wsv, wsh, wrv, wrh, be)


def _final_body(vp, h, p0, p1, wnv, wnh, wna, bn, wd, bd, vn_ref):
    agg = p0[...] + p1[...]
    vo = (_dot(vp[...], wnv[...]) + _dot(h[...], wnh[...])
          + _dot(agg, wna[...]) + bn[...])
    hn = jnp.maximum(vo, 0.0)
    vn_ref[...] = vp[...] + _dot(hn, wd[...]) + bd[...]


def _final_update(npad, vp, h, p0, p1, wnv, wnh, wna, bn, wd, bd):
    grid = npad // NODE_BLK
    full = lambda shape: pl.BlockSpec(shape, lambda i: (0, 0))
    row = lambda w: pl.BlockSpec((NODE_BLK, w), lambda i: (i, 0))
    return pl.pallas_call(
        _final_body,
        grid=(grid,),
        in_specs=[row(2), row(H), row(H), row(H),
                  full((2, H)), full((H, H)), full((H, H)), full((1, H)),
                  full((H, 2)), full((1, 2))],
        out_specs=row(2),
        out_shape=jax.ShapeDtypeStruct((npad, 2), jnp.float32),
    )(vp, h, p0, p1, wnv, wnh, wna, bn, wd, bd)


def _edge_stage(npad, rows, a, b, c, sr):
    rows_per_core = rows // NUM_SC
    nchunk = rows_per_core // NUM_SUBCORES
    agg_rows_sub = npad // NUM_SUBCORES
    z_full = agg_rows_sub // IDX_W
    z_rem = agg_rows_sub % IDX_W
    mesh = plsc.VectorSubcoreMesh(core_axis_name="c", subcore_axis_name="s")

    @functools.partial(
        pl.kernel,
        out_type=jax.ShapeDtypeStruct((NUM_SC, npad, H), jnp.float32),
        mesh=mesh,
        compiler_params=pltpu.CompilerParams(use_tc_tiling_on_sc=False),
        scratch_types=[
            pltpu.VMEM_SHARED((npad, H), jnp.float32),
            pltpu.VMEM((4, 2, IDX_W), jnp.int32),
            pltpu.VMEM((1, IDX_W), jnp.int32),
            pltpu.VMEM((2, IDX_W, H), jnp.float32),
            pltpu.VMEM((2, IDX_W, H), jnp.float32),
            pltpu.VMEM((2, IDX_W, H), jnp.float32),
            pltpu.VMEM((IDX_W, H), jnp.float32),
            pltpu.SemaphoreType.DMA,
            pltpu.SemaphoreType.DMA,
        ],
    )
    def k(a_hbm, b_hbm, c_hbm, sr_hbm, p_hbm,
          agg, srbuf, ridxs, abuf, bbuf, cbuf, mbuf, sem0, sem1):
        ci = lax.axis_index("c")
        si = lax.axis_index("s")
        sems = (sem0, sem1)
        zvec = jnp.zeros((16,), jnp.float32)

        @pl.loop(0, IDX_W)
        def _(i):
            mbuf[i, pl.ds(0, 16)] = zvec
            mbuf[i, pl.ds(16, 16)] = zvec

        z0 = si * agg_rows_sub

        @pl.loop(0, z_full)
        def _(kk):
            pltpu.sync_copy(mbuf, agg.at[pl.ds(z0 + kk * IDX_W, IDX_W)])

        if z_rem:
            pltpu.sync_copy(mbuf.at[pl.ds(0, z_rem)],
                            agg.at[pl.ds(z0 + z_full * IDX_W, z_rem)])
        plsc.subcore_barrier()

        base = ci * rows_per_core + si * nchunk

        def issue(row, rr, ns):
            sem = sems[ns]
            pltpu.async_copy(a_hbm.at[srbuf.at[rr, 0]], abuf.at[ns], sem)
            pltpu.async_copy(b_hbm.at[srbuf.at[rr, 1]], bbuf.at[ns], sem)
            pltpu.async_copy(c_hbm.at[pl.ds(row * IDX_W, IDX_W)],
                             cbuf.at[ns], sem)

        def wait(ns):
            sem = sems[ns]
            pltpu.make_async_copy(a_hbm.at[pl.ds(0, IDX_W)], abuf.at[ns],
                                  sem).wait()
            pltpu.make_async_copy(b_hbm.at[pl.ds(0, IDX_W)], bbuf.at[ns],
                                  sem).wait()
            pltpu.make_async_copy(c_hbm.at[pl.ds(0, IDX_W)], cbuf.at[ns],
                                  sem).wait()

        pltpu.sync_copy(sr_hbm.at[pl.ds(base, 4)], srbuf)
        issue(base, 0, 0)

        @pl.loop(0, nchunk, step=4)
        def _(t):
            for pp in range(4):
                ns = pp % 2
                wait(ns)
                for j in range(IDX_W // 16):
                    ridxs[0, pl.ds(j * 16, 16)] = srbuf[pp, 1,
                                                        pl.ds(j * 16, 16)]
                if pp == 3:
                    @pl.when(t + 4 < nchunk)
                    def _():
                        pltpu.sync_copy(sr_hbm.at[pl.ds(base + t + 4, 4)],
                                        srbuf)
                        issue(base + t + 4, 0, 1 - ns)
                else:
                    issue(base + t + pp + 1, pp + 1, 1 - ns)

                @pl.loop(0, IDX_W)
                def _(i):
                    mbuf[i, pl.ds(0, 16)] = jnp.maximum(
                        abuf[ns, i, pl.ds(0, 16)] + bbuf[ns, i, pl.ds(0, 16)]
                        + cbuf[ns, i, pl.ds(0, 16)], 0.0)
                    mbuf[i, pl.ds(16, 16)] = jnp.maximum(
                        abuf[ns, i, pl.ds(16, 16)]
                        + bbuf[ns, i, pl.ds(16, 16)]
                        + cbuf[ns, i, pl.ds(16, 16)], 0.0)

                pltpu.sync_copy(mbuf, agg.at[ridxs.at[0]], add=True)

        plsc.subcore_barrier()
        pltpu.sync_copy(
            agg.at[pl.ds(si * agg_rows_sub, agg_rows_sub)],
            p_hbm.at[ci, pl.ds(si * agg_rows_sub, agg_rows_sub)])

    return k(a, b, c, sr)


def kernel(P_Q_inj, senders, receivers, edge_features, W_in, b_in,
           We0, be0, Wn0, bn0, Wd0, bd0,
           We1, be1, Wn1, bn1, Wd1, bd1,
           We2, be2, Wn2, bn2, Wd2, bd2):
    f32 = jnp.float32
    n = P_Q_inj.shape[0]
    e = senders.shape[0]
    d_v = 2 + H
    npad = _ceil_to(n + 1, NODE_BLK)
    epad = _ceil_to(e, IDX_W * IDX_W)
    rows = epad // IDX_W

    pq = jnp.zeros((npad, 2), f32).at[:n].set(P_Q_inj)
    s2 = jnp.zeros((epad,), jnp.int32).at[:e].set(senders).reshape(rows, IDX_W)
    r2 = jnp.full((epad,), n, jnp.int32).at[:e].set(receivers).reshape(
        rows, IDX_W)
    sr = jnp.stack([s2, r2], axis=1)
    ef = jnp.zeros((epad, D_EDGE), f32).at[:e].set(edge_features)

    def split_we(we):
        return (we[0:2], we[2:d_v], we[d_v:d_v + 2], we[d_v + 2:2 * d_v],
                we[2 * d_v:])

    layers = []
    for (we, be, wn, bn, wd, bd) in ((We0, be0, Wn0, bn0, Wd0, bd0),
                                     (We1, be1, Wn1, bn1, Wd1, bd1),
                                     (We2, be2, Wn2, bn2, Wd2, bd2)):
        wsv, wsh, wrv, wrh, wee = split_we(we)
        layers.append(dict(
            wsv=wsv, wsh=wsh, wrv=wrv, wrh=wrh, wee=wee,
            be=be.reshape(1, H),
            wnv=wn[0:2], wnh=wn[2:d_v], wna=wn[d_v:],
            bn=bn.reshape(1, H), wd=wd, bd=bd.reshape(1, 2)))

    c_arrs = [_edge_c(epad, ef, lay["wee"]) for lay in layers]

    h, a, b = _prologue(npad, pq, W_in, b_in.reshape(1, H),
                        layers[0]["wsv"], layers[0]["wsh"],
                        layers[0]["wrv"], layers[0]["wrh"], layers[0]["be"])
    vp = jnp.zeros((npad, 2), f32).at[:, 0].set(1.0)

    for li, lay in enumerate(layers):
        p = _edge_stage(npad, rows, a, b, c_arrs[li], sr)
        if li < 2:
            nxt = layers[li + 1]
            vp, h, a, b = _node_update(
                npad, vp, h, p[0], p[1],
                lay["wnv"], lay["wnh"], lay["wna"], lay["bn"],
                lay["wd"], lay["bd"],
                nxt["wsv"], nxt["wsh"], nxt["wrv"], nxt["wrh"], nxt["be"])
        else:
            vp = _final_update(
                npad, vp, h, p[0], p[1],
                lay["wnv"], lay["wnh"], lay["wna"], lay["bn"],
                lay["wd"], lay["bd"])

    return vp[:n]


# speedup vs baseline: 5.8415x; 1.3985x over previous
import functools

import jax
import jax.numpy as jnp
from jax import lax
from jax.experimental import pallas as pl
from jax.experimental.pallas import tpu as pltpu
from jax.experimental.pallas import tpu_sc as plsc

H = 32
D_EDGE = 4
NODE_BLK = 1000
C_BLK = 1000
IDX_W = 128
CW = 1024
NUM_SC = 2
NUM_SUBCORES = 16
NUM_W = NUM_SC * NUM_SUBCORES


def _dot(x, w):
    return jnp.dot(x, w, preferred_element_type=jnp.float32)


def _prologue_body(pq, win, bin_, wsv, wsh, wrv, wrh, be, h_ref, a_ref, b_ref):
    h = _dot(pq[...], win[...]) + bin_[...]
    h_ref[...] = h
    a_ref[...] = (_dot(h, wsh[...]) + wsv[...][0:1, :]).astype(jnp.bfloat16)
    b_ref[...] = (_dot(h, wrh[...]) + wrv[...][0:1, :]
                  + be[...]).astype(jnp.bfloat16)


def _prologue(n, pq, win, bin_, wsv, wsh, wrv, wrh, be):
    grid = n // NODE_BLK
    full = lambda shape: pl.BlockSpec(shape, lambda i: (0, 0))
    row = lambda w: pl.BlockSpec((NODE_BLK, w), lambda i: (i, 0))
    outf = jax.ShapeDtypeStruct((n, H), jnp.float32)
    outb = jax.ShapeDtypeStruct((n, H), jnp.bfloat16)
    return pl.pallas_call(
        _prologue_body,
        grid=(grid,),
        in_specs=[row(2), full((2, H)), full((1, H)), full((2, H)),
                  full((H, H)), full((2, H)), full((H, H)), full((1, H))],
        out_specs=[row(H), row(H), row(H)],
        out_shape=[outf, outb, outb],
    )(pq, win, bin_, wsv, wsh, wrv, wrh, be)


def _edge_c_body(x, w2, c_ref):
    c_ref[...] = _dot(x[...], w2[...])


def _edge_c(crows, x, w2):
    grid = crows // C_BLK
    return pl.pallas_call(
        _edge_c_body,
        grid=(grid,),
        in_specs=[pl.BlockSpec((C_BLK, 128), lambda i: (i, 0)),
                  pl.BlockSpec((128, CW), lambda i: (0, 0))],
        out_specs=pl.BlockSpec((C_BLK, CW), lambda i: (i, 0)),
        out_shape=jax.ShapeDtypeStruct((crows, CW), jnp.float32),
    )(x, w2)


def _node_body(vp, h, p0, p1, wnall, bn, wd, bd, wab, be,
               vn_ref, hn_ref, an_ref, bn_ref):
    agg = p0[...][0].astype(jnp.float32) + p1[...][0].astype(jnp.float32)
    x = jnp.concatenate([vp[...], h[...], agg], axis=1)
    vo = _dot(x, wnall[...]) + bn[...]
    hn = jnp.maximum(vo, 0.0)
    vn = vp[...] + _dot(hn, wd[...]) + bd[...]
    vn_ref[...] = vn
    hn_ref[...] = hn
    d = _dot(jnp.concatenate([vn, hn], axis=1), wab[...])
    an_ref[...] = d[:, 0:H].astype(jnp.bfloat16)
    bn_ref[...] = (d[:, H:2 * H] + be[...]).astype(jnp.bfloat16)


def _node_update(n, vp, h, p, wnall, bn, wd, bd, wab, be):
    grid = n // NODE_BLK
    full = lambda shape: pl.BlockSpec(shape, lambda i: (0, 0))
    row = lambda w: pl.BlockSpec((NODE_BLK, w), lambda i: (i, 0))
    outh = jax.ShapeDtypeStruct((n, H), jnp.float32)
    outb = jax.ShapeDtypeStruct((n, H), jnp.bfloat16)
    outv = jax.ShapeDtypeStruct((n, 2), jnp.float32)
    p0row = pl.BlockSpec((1, NODE_BLK, H), lambda i: (0, i, 0))
    p1row = pl.BlockSpec((1, NODE_BLK, H), lambda i: (1, i, 0))
    return pl.pallas_call(
        _node_body,
        grid=(grid,),
        in_specs=[row(2), row(H), p0row, p1row,
                  full((2 + 2 * H, H)), full((1, H)),
                  full((H, 2)), full((1, 2)),
                  full((2 + H, 2 * H)), full((1, H))],
        out_specs=[row(2), row(H), row(H), row(H)],
        out_shape=[outv, outh, outb, outb],
    )(vp, h, p, p, wnall, bn, wd, bd, wab, be)


def _final_body(vp, h, p0, p1, wnall, bn, wd, bd, vn_ref):
    agg = p0[...][0].astype(jnp.float32) + p1[...][0].astype(jnp.float32)
    x = jnp.concatenate([vp[...], h[...], agg], axis=1)
    vo = _dot(x, wnall[...]) + bn[...]
    hn = jnp.maximum(vo, 0.0)
    vn_ref[...] = vp[...] + _dot(hn, wd[...]) + bd[...]


def _final_update(n, vp, h, p, wnall, bn, wd, bd):
    grid = n // NODE_BLK
    full = lambda shape: pl.BlockSpec(shape, lambda i: (0, 0))
    row = lambda w: pl.BlockSpec((NODE_BLK, w), lambda i: (i, 0))
    p0row = pl.BlockSpec((1, NODE_BLK, H), lambda i: (0, i, 0))
    p1row = pl.BlockSpec((1, NODE_BLK, H), lambda i: (1, i, 0))
    return pl.pallas_call(
        _final_body,
        grid=(grid,),
        in_specs=[row(2), row(H), p0row, p1row,
                  full((2 + 2 * H, H)), full((1, H)),
                  full((H, 2)), full((1, 2))],
        out_specs=row(2),
        out_shape=jax.ShapeDtypeStruct((n, 2), jnp.float32),
    )(vp, h, p, p, wnall, bn, wd, bd)


def _edge_stage(n, e, a, b, c, s_idx, r_idx):
    chunks = e // IDX_W
    per_w = chunks // NUM_W
    extra = chunks % NUM_W
    agg_rows_sub = n // NUM_SUBCORES
    z_full = agg_rows_sub // IDX_W
    z_rem = agg_rows_sub % IDX_W
    mesh = plsc.VectorSubcoreMesh(core_axis_name="c", subcore_axis_name="s")

    @functools.partial(
        pl.kernel,
        out_type=jax.ShapeDtypeStruct((NUM_SC, n, H), jnp.bfloat16),
        mesh=mesh,
        compiler_params=pltpu.CompilerParams(use_tc_tiling_on_sc=False,
                                             needs_layout_passes=False),
        scratch_types=[
            pltpu.VMEM_SHARED((n, H), jnp.bfloat16),
            pltpu.VMEM((4 * IDX_W,), jnp.int32),
            pltpu.VMEM((4 * IDX_W,), jnp.int32),
            pltpu.VMEM((2, IDX_W), jnp.int32),
            pltpu.VMEM((2, IDX_W, H), jnp.bfloat16),
            pltpu.VMEM((2, IDX_W, H), jnp.bfloat16),
            pltpu.VMEM((2, IDX_W * H // CW, CW), jnp.float32),
            pltpu.VMEM((2, IDX_W, H), jnp.bfloat16),
            pltpu.SemaphoreType.DMA,
            pltpu.SemaphoreType.DMA,
            pltpu.SemaphoreType.DMA,
        ],
    )
    def k(a_hbm, b_hbm, c_hbm, s_hbm, r_hbm, p_hbm,
          agg, sbuf, rbuf, ridxs, abuf, bbuf, cbuf, mbuf, sem0, sem1, sem_sc):
        ci = lax.axis_index("c")
        si = lax.axis_index("s")
        wid = ci * NUM_SUBCORES + si
        sems = (sem0, sem1)
        zvec = jnp.zeros((2 * 16,), jnp.bfloat16)
        c_rows = IDX_W * H // CW

        @pl.loop(0, IDX_W)
        def _(i):
            mbuf[0, i, :] = zvec

        z0 = si * agg_rows_sub

        @pl.loop(0, z_full)
        def _(kk):
            pltpu.async_copy(mbuf.at[0],
                             agg.at[pl.ds(z0 + kk * IDX_W, IDX_W)], sem0)

        if z_rem:
            pltpu.async_copy(mbuf.at[0, pl.ds(0, z_rem)],
                             agg.at[pl.ds(z0 + z_full * IDX_W, z_rem)], sem0)

        @pl.loop(0, z_full)
        def _(kk):
            pltpu.make_async_copy(mbuf.at[0], agg.at[pl.ds(z0, IDX_W)],
                                  sem0).wait()

        if z_rem:
            pltpu.make_async_copy(mbuf.at[0, pl.ds(0, z_rem)],
                                  agg.at[pl.ds(z0, z_rem)], sem0).wait()
        plsc.subcore_barrier()

        base = wid * per_w + jnp.minimum(wid, extra)
        cnt = jnp.where(wid < extra, per_w + 1, per_w)
        nfull = (cnt // 4) * 4

        def issue(chunk, rr, ns):
            sem = sems[ns]
            pltpu.async_copy(a_hbm.at[sbuf.at[pl.ds(rr * IDX_W, IDX_W)]],
                             abuf.at[ns], sem)
            pltpu.async_copy(b_hbm.at[rbuf.at[pl.ds(rr * IDX_W, IDX_W)]],
                             bbuf.at[ns], sem)
            pltpu.async_copy(c_hbm.at[pl.ds(chunk * c_rows, c_rows)],
                             cbuf.at[ns], sem)

        def wait(ns):
            sem = sems[ns]
            pltpu.make_async_copy(a_hbm.at[pl.ds(0, IDX_W)], abuf.at[ns],
                                  sem).wait()
            pltpu.make_async_copy(b_hbm.at[pl.ds(0, IDX_W)], bbuf.at[ns],
                                  sem).wait()
            pltpu.make_async_copy(c_hbm.at[pl.ds(0, c_rows)], cbuf.at[ns],
                                  sem).wait()

        def load_idx_block(chunk0):
            pltpu.sync_copy(s_hbm.at[pl.ds(chunk0 * IDX_W, 4 * IDX_W)], sbuf)
            pltpu.sync_copy(r_hbm.at[pl.ds(chunk0 * IDX_W, 4 * IDX_W)], rbuf)

        def stash_ridx(rr, ns):
            for j in range(IDX_W // 16):
                ridxs[ns, pl.ds(j * 16, 16)] = rbuf[pl.ds(rr * IDX_W + j * 16,
                                                          16)]

        def compute(ns):
            @plsc.parallel_loop(0, IDX_W, step=8)
            def _(i):
                crow = i // (CW // H)
                cbase = (i - crow * (CW // H)) * H
                for u in range(8):
                    a0, a1 = plsc.unpack(
                        abuf[ns, i + u, :],
                        format=plsc.PackFormat.INTERLEAVED,
                        preferred_element_type=jnp.float32)
                    b0, b1 = plsc.unpack(
                        bbuf[ns, i + u, :],
                        format=plsc.PackFormat.INTERLEAVED,
                        preferred_element_type=jnp.float32)
                    c0 = cbuf[ns, crow, pl.ds(cbase + u * H, 16)]
                    c1 = cbuf[ns, crow, pl.ds(cbase + u * H + 16, 16)]
                    m0 = jnp.maximum(a0 + b0 + c0, 0.0)
                    m1 = jnp.maximum(a1 + b1 + c1, 0.0)
                    mbuf[ns, i + u, :] = plsc.pack(
                        m0, m1, format=plsc.PackFormat.INTERLEAVED)

        def wait_scatter(ns):
            pltpu.make_async_copy(mbuf.at[ns], agg.at[ridxs.at[ns]],
                                  sem_sc).wait()

        load_idx_block(base)
        issue(base, 0, 0)

        @pl.loop(0, nfull, step=4)
        def _(t):
            for pp in range(4):
                ns = pp % 2
                wait(ns)
                if pp < 2:
                    @pl.when(t > 0)
                    def _():
                        wait_scatter(ns)
                else:
                    wait_scatter(ns)
                stash_ridx(pp, ns)
                if pp == 3:
                    @pl.when(t + 4 < nfull)
                    def _():
                        load_idx_block(base + t + 4)
                        issue(base + t + 4, 0, 1 - ns)
                else:
                    issue(base + t + pp + 1, pp + 1, 1 - ns)
                compute(ns)
                pltpu.async_copy(mbuf.at[ns], agg.at[ridxs.at[ns]], sem_sc,
                                 add=True)

        @pl.when(nfull > 0)
        def _():
            wait_scatter(0)
            wait_scatter(1)

        @pl.loop(0, cnt - nfull)
        def _(kk):
            chunk = base + nfull + kk
            pltpu.sync_copy(s_hbm.at[pl.ds(chunk * IDX_W, IDX_W)],
                            sbuf.at[pl.ds(0, IDX_W)])
            pltpu.sync_copy(r_hbm.at[pl.ds(chunk * IDX_W, IDX_W)],
                            rbuf.at[pl.ds(0, IDX_W)])
            issue(chunk, 0, 0)
            wait(0)
            stash_ridx(0, 0)
            compute(0)
            pltpu.sync_copy(mbuf.at[0], agg.at[ridxs.at[0]], add=True)

        plsc.subcore_barrier()
        pltpu.sync_copy(
            agg.at[pl.ds(si * agg_rows_sub, agg_rows_sub)],
            p_hbm.at[ci, pl.ds(si * agg_rows_sub, agg_rows_sub)])

    return k(a, b, c, s_idx, r_idx)


def kernel(P_Q_inj, senders, receivers, edge_features, W_in, b_in,
           We0, be0, Wn0, bn0, Wd0, bd0,
           We1, be1, Wn1, bn1, Wd1, bd1,
           We2, be2, Wn2, bn2, Wd2, bd2):
    f32 = jnp.float32
    n = P_Q_inj.shape[0]
    e = senders.shape[0]
    d_v = 2 + H

    efv = edge_features.reshape(e * D_EDGE // 128, 128)
    crows = e * H // CW

    def split_we(we):
        return (we[0:2], we[2:d_v], we[d_v:d_v + 2], we[d_v + 2:2 * d_v],
                we[2 * d_v:])

    ileave = jnp.array([v for i in range(H // 2) for v in (i, H // 2 + i)],
                       dtype=jnp.int32)

    layers = []
    for (we, be, wn, bn, wd, bd) in ((We0, be0, Wn0, bn0, Wd0, bd0),
                                     (We1, be1, Wn1, bn1, Wd1, bd1),
                                     (We2, be2, Wn2, bn2, Wd2, bd2)):
        wsv, wsh, wrv, wrh, wee = split_we(we)
        w2 = jnp.zeros((128, CW), f32)
        for q in range(32):
            w2 = w2.at[q * D_EDGE:(q + 1) * D_EDGE,
                       q * H:(q + 1) * H].set(wee)
        wsv, wsh = wsv[:, ileave], wsh[:, ileave]
        wrv, wrh = wrv[:, ileave], wrh[:, ileave]
        layers.append(dict(
            wsv=wsv, wsh=wsh, wrv=wrv, wrh=wrh, wee=w2,
            be=be.reshape(1, H)[:, ileave],
            wnall=jnp.concatenate([wn[0:2], wn[2:d_v], wn[d_v:][ileave]], 0),
            wab=jnp.concatenate(
                [jnp.concatenate([wsv, wrv], 1),
                 jnp.concatenate([wsh, wrh], 1)], 0),
            bn=bn.reshape(1, H), wd=wd, bd=bd.reshape(1, 2)))

    c_arrs = [_edge_c(crows, efv, lay["wee"]) for lay in layers]

    h, a, b = _prologue(n, P_Q_inj, W_in, b_in.reshape(1, H),
                        layers[0]["wsv"], layers[0]["wsh"],
                        layers[0]["wrv"], layers[0]["wrh"], layers[0]["be"])
    vp = jnp.zeros((n, 2), f32).at[:, 0].set(1.0)

    for li, lay in enumerate(layers):
        p = _edge_stage(n, e, a, b, c_arrs[li], senders, receivers)
        if li < 2:
            nxt = layers[li + 1]
            vp, h, a, b = _node_update(
                n, vp, h, p,
                lay["wnall"], lay["bn"], lay["wd"], lay["bd"],
                nxt["wab"], nxt["be"])
        else:
            vp = _final_update(
                n, vp, h, p,
                lay["wnall"], lay["bn"], lay["wd"], lay["bd"])

    return vp
